# baseline (speedup 1.0000x reference)
import functools

import jax
import jax.numpy as jnp
from jax import lax
from jax.experimental import pallas as pl
from jax.experimental.pallas import tpu as pltpu
from jax.experimental.pallas import tpu_sc as plsc

TAG = 32
_SIGN = -(2 ** 31)

_NC = 2
_NS = 16
_NW = _NC * _NS

_CH = 16


def _tc_body(x_ref, wt_ref, b_ref, kc_ref, t_ref):
    x = x_ref[...]
    xc = x - jnp.mean(x, axis=1, keepdims=True)
    kc = jnp.dot(xc, wt_ref[...], preferred_element_type=jnp.float32)
    kc = kc + b_ref[...]

    b = lax.bitcast_convert_type(kc, jnp.int32)
    s = jnp.where(b >= 0, b, b ^ jnp.int32(0x7FFFFFFF))

    rows = kc.shape[0]
    ones = jnp.ones((kc.shape[1], 1), jnp.float32)

    def step(i, cur):
        bit = lax.shift_left(jnp.int32(1), jnp.int32(31) - i)
        cand_u = cur | bit
        cand_s = cand_u ^ jnp.int32(_SIGN)
        mask = jnp.where(s >= cand_s, 1.0, 0.0)
        cnt = jnp.dot(mask, ones, preferred_element_type=jnp.float32)
        return jnp.where(cnt >= float(TAG), cand_u, cur)

    cur = lax.fori_loop(0, 32, step, jnp.zeros((rows, 1), jnp.int32))
    t_s = cur ^ jnp.int32(_SIGN)
    t_b = jnp.where(t_s >= 0, t_s, t_s ^ jnp.int32(0x7FFFFFFF))
    kc_ref[...] = kc
    t_ref[...] = jnp.broadcast_to(
        lax.bitcast_convert_type(t_b, jnp.float32), t_ref.shape)


def _project_and_threshold(x, wt, b2):
    batch, in_dim = x.shape
    out_dim = wt.shape[1]
    br = min(512, batch)
    return pl.pallas_call(
        _tc_body,
        grid=(batch // br,),
        in_specs=[
            pl.BlockSpec((br, in_dim), lambda i: (i, 0)),
            pl.BlockSpec((in_dim, out_dim), lambda i: (0, 0)),
            pl.BlockSpec((1, out_dim), lambda i: (0, 0)),
        ],
        out_specs=[
            pl.BlockSpec((br, out_dim), lambda i: (i, 0)),
            pl.BlockSpec((br, 16), lambda i: (i, 0)),
        ],
        out_shape=[
            jax.ShapeDtypeStruct((batch, out_dim), jnp.float32),
            jax.ShapeDtypeStruct((batch, 16), jnp.float32),
        ],
        compiler_params=pltpu.CompilerParams(
            dimension_semantics=("parallel",)
        ),
    )(x, wt, b2)


def _sc_mask(kc, t, batch, out_dim):
    rows_per_w = batch // _NW
    chunks = rows_per_w // _CH
    mesh = plsc.VectorSubcoreMesh(
        core_axis_name="c", subcore_axis_name="s",
        num_cores=_NC, num_subcores=_NS,
    )

    @functools.partial(
        pl.kernel,
        out_type=jax.ShapeDtypeStruct((batch, out_dim), jnp.float32),
        mesh=mesh,
        scratch_types=[
            pltpu.VMEM((_CH, out_dim), jnp.float32),
            pltpu.VMEM((rows_per_w, 16), jnp.float32),
        ],
    )
    def k(kc_hbm, t_hbm, out_hbm, buf, t_v):
        wid = lax.axis_index("s") * _NC + lax.axis_index("c")
        row0 = wid * rows_per_w
        pltpu.sync_copy(t_hbm.at[pl.ds(row0, rows_per_w)], t_v)

        def chunk_body(c, _):
            base = row0 + c * _CH
            pltpu.sync_copy(kc_hbm.at[pl.ds(base, _CH)], buf)
            for r in range(_CH):
                tvec = t_v[c * _CH + r, :]

                def slice_body(i, _, r=r, tvec=tvec):
                    for u in range(8):
                        off = i * 128 + u * 16
                        v = buf[r, pl.ds(off, 16)]
                        buf[r, pl.ds(off, 16)] = jnp.where(
                            v >= tvec, v, jnp.zeros((16,), jnp.float32))
                    return 0

                lax.fori_loop(0, out_dim // 128, slice_body, 0)
            pltpu.sync_copy(buf, out_hbm.at[pl.ds(base, _CH)])
            return 0

        lax.fori_loop(0, chunks, chunk_body, 0)

    return k(kc, t)


@jax.jit
def kernel(x, W, b):
    batch, in_dim = x.shape
    out_dim = W.shape[0]
    wt = W.T
    b2 = b.reshape(1, out_dim)
    kc, t = _project_and_threshold(x, wt, b2)
    return _sc_mask(kc, t, batch, out_dim)


# speedup vs baseline: 1.2720x; 1.2720x over previous
import functools

import jax
import jax.numpy as jnp
from jax import lax
from jax.experimental import pallas as pl
from jax.experimental.pallas import tpu as pltpu
from jax.experimental.pallas import tpu_sc as plsc

TAG = 32
_SIGN = -(2 ** 31)

_NC = 2
_NS = 16
_NW = _NC * _NS

_CH = 8
_NBUF = 4


def _tc_body(x_ref, wt_ref, b_ref, kc_ref, t_ref):
    x = x_ref[...]
    xc = x - jnp.mean(x, axis=1, keepdims=True)
    kc = jnp.dot(xc, wt_ref[...], preferred_element_type=jnp.float32)
    kc = kc + b_ref[...]

    b = lax.bitcast_convert_type(kc, jnp.int32)
    s = jnp.where(b >= 0, b, b ^ jnp.int32(0x7FFFFFFF))

    rows = kc.shape[0]
    def step(i, cur):
        bit = lax.shift_left(jnp.int32(1), jnp.int32(31) - i)
        cand_u = cur | bit
        cand_s = cand_u ^ jnp.int32(_SIGN)
        cnt = jnp.sum((s >= cand_s).astype(jnp.int32), axis=1, keepdims=True)
        return jnp.where(cnt >= TAG, cand_u, cur)

    cur = lax.fori_loop(0, 32, step, jnp.zeros((rows, 1), jnp.int32))
    t_s = cur ^ jnp.int32(_SIGN)
    t_b = jnp.where(t_s >= 0, t_s, t_s ^ jnp.int32(0x7FFFFFFF))
    kc_ref[...] = kc
    t_ref[...] = jnp.broadcast_to(
        lax.bitcast_convert_type(t_b, jnp.float32), t_ref.shape)


def _project_and_threshold(x, wt, b2):
    batch, in_dim = x.shape
    out_dim = wt.shape[1]
    br = min(512, batch)
    return pl.pallas_call(
        _tc_body,
        grid=(batch // br,),
        in_specs=[
            pl.BlockSpec((br, in_dim), lambda i: (i, 0)),
            pl.BlockSpec((in_dim, out_dim), lambda i: (0, 0)),
            pl.BlockSpec((1, out_dim), lambda i: (0, 0)),
        ],
        out_specs=[
            pl.BlockSpec((br, out_dim), lambda i: (i, 0)),
            pl.BlockSpec((br, 16), lambda i: (i, 0)),
        ],
        out_shape=[
            jax.ShapeDtypeStruct((batch, out_dim), jnp.float32),
            jax.ShapeDtypeStruct((batch, 16), jnp.float32),
        ],
        compiler_params=pltpu.CompilerParams(
            dimension_semantics=("parallel",)
        ),
    )(x, wt, b2)


def _sc_mask(kc, t, batch, out_dim):
    rows_per_w = batch // _NW
    chunks = rows_per_w // _CH
    mesh = plsc.VectorSubcoreMesh(
        core_axis_name="c", subcore_axis_name="s",
        num_cores=_NC, num_subcores=_NS,
    )

    @functools.partial(
        pl.kernel,
        out_type=jax.ShapeDtypeStruct((batch, out_dim), jnp.float32),
        mesh=mesh,
        scratch_types=(
            [pltpu.VMEM((_CH, out_dim), jnp.float32)] * _NBUF
            + [pltpu.VMEM((rows_per_w, 16), jnp.float32)]
            + [pltpu.SemaphoreType.DMA] * (2 * _NBUF)
        ),
    )
    def k(kc_hbm, t_hbm, out_hbm, *refs):
        bufs = refs[:_NBUF]
        t_v = refs[_NBUF]
        sin = refs[_NBUF + 1:2 * _NBUF + 1]
        sout = refs[2 * _NBUF + 1:]
        wid = lax.axis_index("s") * _NC + lax.axis_index("c")
        row0 = wid * rows_per_w
        pltpu.sync_copy(t_hbm.at[pl.ds(row0, rows_per_w)], t_v)

        def src(c):
            return kc_hbm.at[pl.ds(row0 + c * _CH, _CH)]

        def dst(c):
            return out_hbm.at[pl.ds(row0 + c * _CH, _CH)]

        for p in range(_NBUF - 1):
            pltpu.async_copy(src(p), bufs[p], sin[p])

        def group(g, _):
            for p in range(_NBUF):
                c = g * _NBUF + p
                buf, s_i, s_o = bufs[p], sin[p], sout[p]
                pltpu.make_async_copy(src(c), buf, s_i).wait()
                for r in range(_CH):
                    tvec = t_v[c * _CH + r, :]

                    def slice_body(i, _, r=r, tvec=tvec, buf=buf):
                        for u in range(8):
                            off = i * 128 + u * 16
                            v = buf[r, pl.ds(off, 16)]
                            buf[r, pl.ds(off, 16)] = jnp.where(
                                v >= tvec, v, jnp.zeros((16,), jnp.float32))
                        return 0

                    lax.fori_loop(0, out_dim // 128, slice_body, 0)
                pltpu.async_copy(buf, dst(c), s_o)
                q = (p + _NBUF - 1) % _NBUF
                nc = c + _NBUF - 1

                @pl.when(jnp.logical_and(nc < chunks, nc >= _NBUF))
                def _(q=q, c=c):
                    pltpu.make_async_copy(bufs[q], dst(c - 1), sout[q]).wait()

                @pl.when(nc < chunks)
                def _(q=q, nc=nc):
                    pltpu.async_copy(src(nc), bufs[q], sin[q])
            return 0

        lax.fori_loop(0, chunks // _NBUF, group, 0)
        for p in range(_NBUF):
            pltpu.make_async_copy(bufs[p], dst(p), sout[p]).wait()

    return k(kc, t)


@jax.jit
def kernel(x, W, b):
    batch, in_dim = x.shape
    out_dim = W.shape[0]
    wt = W.T
    b2 = b.reshape(1, out_dim)
    kc, t = _project_and_threshold(x, wt, b2)
    return _sc_mask(kc, t, batch, out_dim)


# speedup vs baseline: 1.4000x; 1.1007x over previous
import functools

import jax
import jax.numpy as jnp
from jax import lax
from jax.experimental import pallas as pl
from jax.experimental.pallas import tpu as pltpu
from jax.experimental.pallas import tpu_sc as plsc

TAG = 32
_SIGN = -(2 ** 31)

_NC = 2
_NS = 16
_NW = _NC * _NS

_CH = 8
_NBUF = 4


def _tc_body(x_ref, wt_ref, b_ref, kc_ref, t_ref):
    x = x_ref[...]
    xc = x - jnp.mean(x, axis=1, keepdims=True)
    kc = jnp.dot(xc, wt_ref[...], preferred_element_type=jnp.float32)
    kc = kc + b_ref[...]

    b = lax.bitcast_convert_type(kc, jnp.int32)
    s = jnp.where(b >= 0, b, b ^ jnp.int32(0x7FFFFFFF))

    rows = kc.shape[0]
    def step(i, cur):
        bit = lax.shift_left(jnp.int32(1), jnp.int32(31) - i)
        cand_u = cur | bit
        cand_s = cand_u ^ jnp.int32(_SIGN)
        cnt = jnp.sum((s >= cand_s).astype(jnp.int32), axis=1, keepdims=True)
        return jnp.where(cnt >= TAG, cand_u, cur)

    cur = lax.fori_loop(0, 16, step, jnp.zeros((rows, 1), jnp.int32))
    lo_s = cur ^ jnp.int32(_SIGN)
    hi_s = (cur + jnp.int32(1 << 16)) ^ jnp.int32(_SIGN)
    n_hi = jnp.sum((s >= hi_s).astype(jnp.int32), axis=1, keepdims=True)
    m = TAG - n_hi
    imin = jnp.int32(_SIGN)
    vb = jnp.where((s >= lo_s) & (s < hi_s), s, imin)

    def tail_cond(state):
        _, m_rem, _ = state
        return jnp.any(m_rem > 0)

    def tail_body(state):
        prev, m_rem, tfin = state
        active = m_rem > 0
        mx = jnp.max(jnp.where(vb < prev, vb, imin), axis=1, keepdims=True)
        ceq = jnp.sum((vb == mx).astype(jnp.int32), axis=1, keepdims=True)
        m_new = m_rem - ceq
        done_now = active & (m_new <= 0)
        tfin = jnp.where(done_now, mx, tfin)
        return (jnp.where(active, mx, prev),
                jnp.where(active, m_new, m_rem), tfin)

    imax = jnp.int32(2 ** 31 - 1)
    _, _, t_s = lax.while_loop(
        tail_cond, tail_body,
        (jnp.full((rows, 1), imax), m, jnp.full((rows, 1), imax)))
    t_b = jnp.where(t_s >= 0, t_s, t_s ^ jnp.int32(0x7FFFFFFF))
    kc_ref[...] = kc
    t_ref[...] = jnp.broadcast_to(
        lax.bitcast_convert_type(t_b, jnp.float32), t_ref.shape)


def _project_and_threshold(x, wt, b2):
    batch, in_dim = x.shape
    out_dim = wt.shape[1]
    br = min(512, batch)
    return pl.pallas_call(
        _tc_body,
        grid=(batch // br,),
        in_specs=[
            pl.BlockSpec((br, in_dim), lambda i: (i, 0)),
            pl.BlockSpec((in_dim, out_dim), lambda i: (0, 0)),
            pl.BlockSpec((1, out_dim), lambda i: (0, 0)),
        ],
        out_specs=[
            pl.BlockSpec((br, out_dim), lambda i: (i, 0)),
            pl.BlockSpec((br, 16), lambda i: (i, 0)),
        ],
        out_shape=[
            jax.ShapeDtypeStruct((batch, out_dim), jnp.float32),
            jax.ShapeDtypeStruct((batch, 16), jnp.float32),
        ],
        compiler_params=pltpu.CompilerParams(
            dimension_semantics=("parallel",)
        ),
    )(x, wt, b2)


def _sc_mask(kc, t, batch, out_dim):
    rows_per_w = batch // _NW
    chunks = rows_per_w // _CH
    mesh = plsc.VectorSubcoreMesh(
        core_axis_name="c", subcore_axis_name="s",
        num_cores=_NC, num_subcores=_NS,
    )

    @functools.partial(
        pl.kernel,
        out_type=jax.ShapeDtypeStruct((batch, out_dim), jnp.float32),
        mesh=mesh,
        scratch_types=(
            [pltpu.VMEM((_CH, out_dim), jnp.float32)] * _NBUF
            + [pltpu.VMEM((rows_per_w, 16), jnp.float32)]
            + [pltpu.SemaphoreType.DMA] * (2 * _NBUF)
        ),
    )
    def k(kc_hbm, t_hbm, out_hbm, *refs):
        bufs = refs[:_NBUF]
        t_v = refs[_NBUF]
        sin = refs[_NBUF + 1:2 * _NBUF + 1]
        sout = refs[2 * _NBUF + 1:]
        wid = lax.axis_index("s") * _NC + lax.axis_index("c")
        row0 = wid * rows_per_w
        pltpu.sync_copy(t_hbm.at[pl.ds(row0, rows_per_w)], t_v)

        def src(c):
            return kc_hbm.at[pl.ds(row0 + c * _CH, _CH)]

        def dst(c):
            return out_hbm.at[pl.ds(row0 + c * _CH, _CH)]

        for p in range(_NBUF - 1):
            pltpu.async_copy(src(p), bufs[p], sin[p])

        def group(g, _):
            for p in range(_NBUF):
                c = g * _NBUF + p
                buf, s_i, s_o = bufs[p], sin[p], sout[p]
                pltpu.make_async_copy(src(c), buf, s_i).wait()
                for r in range(_CH):
                    tvec = t_v[c * _CH + r, :]

                    def slice_body(i, _, r=r, tvec=tvec, buf=buf):
                        for u in range(8):
                            off = i * 128 + u * 16
                            v = buf[r, pl.ds(off, 16)]
                            buf[r, pl.ds(off, 16)] = jnp.where(
                                v >= tvec, v, jnp.zeros((16,), jnp.float32))
                        return 0

                    lax.fori_loop(0, out_dim // 128, slice_body, 0)
                pltpu.async_copy(buf, dst(c), s_o)
                q = (p + _NBUF - 1) % _NBUF
                nc = c + _NBUF - 1

                @pl.when(jnp.logical_and(nc < chunks, nc >= _NBUF))
                def _(q=q, c=c):
                    pltpu.make_async_copy(bufs[q], dst(c - 1), sout[q]).wait()

                @pl.when(nc < chunks)
                def _(q=q, nc=nc):
                    pltpu.async_copy(src(nc), bufs[q], sin[q])
            return 0

        lax.fori_loop(0, chunks // _NBUF, group, 0)
        for p in range(_NBUF):
            pltpu.make_async_copy(bufs[p], dst(p), sout[p]).wait()

    return k(kc, t)


@jax.jit
def kernel(x, W, b):
    batch, in_dim = x.shape
    out_dim = W.shape[0]
    wt = W.T
    b2 = b.reshape(1, out_dim)
    kc, t = _project_and_threshold(x, wt, b2)
    return _sc_mask(kc, t, batch, out_dim)


# speedup vs baseline: 1.4163x; 1.0116x over previous
import functools

import jax
import jax.numpy as jnp
from jax import lax
from jax.experimental import pallas as pl
from jax.experimental.pallas import tpu as pltpu
from jax.experimental.pallas import tpu_sc as plsc

TAG = 32
_SIGN = -(2 ** 31)

_NC = 2
_NS = 16
_NW = _NC * _NS

_CH = 8
_NBUF = 4


def _tc_body(x_ref, wt_ref, b_ref, kc_ref, t_ref):
    x = x_ref[...]
    xc = x - jnp.mean(x, axis=1, keepdims=True)
    kc = jnp.dot(xc, wt_ref[...], preferred_element_type=jnp.float32)
    kc = kc + b_ref[...]

    b = lax.bitcast_convert_type(kc, jnp.int32)
    s = jnp.where(b >= 0, b, b ^ jnp.int32(0x7FFFFFFF))

    rows = kc.shape[0]
    def step(i, cur):
        bit = lax.shift_left(jnp.int32(1), jnp.int32(31) - i)
        cand_u = cur | bit
        cand_s = cand_u ^ jnp.int32(_SIGN)
        cnt = jnp.sum((s >= cand_s).astype(jnp.int32), axis=1, keepdims=True)
        return jnp.where(cnt >= TAG, cand_u, cur)

    cur = lax.fori_loop(0, 16, step, jnp.zeros((rows, 1), jnp.int32))
    lo_s = cur ^ jnp.int32(_SIGN)
    hi_s = (cur + jnp.int32(1 << 16)) ^ jnp.int32(_SIGN)
    n_hi = jnp.sum((s >= hi_s).astype(jnp.int32), axis=1, keepdims=True)
    m = TAG - n_hi
    imin = jnp.int32(_SIGN)
    vb = jnp.where((s >= lo_s) & (s < hi_s), s, imin)

    def tail_cond(state):
        _, m_rem, _ = state
        return jnp.any(m_rem > 0)

    def tail_body(state):
        prev, m_rem, tfin = state
        active = m_rem > 0
        mx = jnp.max(jnp.where(vb < prev, vb, imin), axis=1, keepdims=True)
        ceq = jnp.sum((vb == mx).astype(jnp.int32), axis=1, keepdims=True)
        m_new = m_rem - ceq
        done_now = active & (m_new <= 0)
        tfin = jnp.where(done_now, mx, tfin)
        return (jnp.where(active, mx, prev),
                jnp.where(active, m_new, m_rem), tfin)

    imax = jnp.int32(2 ** 31 - 1)
    _, _, t_s = lax.while_loop(
        tail_cond, tail_body,
        (jnp.full((rows, 1), imax), m, jnp.full((rows, 1), imax)))
    t_b = jnp.where(t_s >= 0, t_s, t_s ^ jnp.int32(0x7FFFFFFF))
    kc_ref[...] = kc
    t_ref[...] = jnp.broadcast_to(
        lax.bitcast_convert_type(t_b, jnp.float32), t_ref.shape)


def _project_and_threshold(x, wt, b2):
    batch, in_dim = x.shape
    out_dim = wt.shape[1]
    br = min(1024, batch)
    return pl.pallas_call(
        _tc_body,
        grid=(batch // br,),
        in_specs=[
            pl.BlockSpec((br, in_dim), lambda i: (i, 0)),
            pl.BlockSpec((in_dim, out_dim), lambda i: (0, 0)),
            pl.BlockSpec((1, out_dim), lambda i: (0, 0)),
        ],
        out_specs=[
            pl.BlockSpec((br, out_dim), lambda i: (i, 0)),
            pl.BlockSpec((br, 16), lambda i: (i, 0)),
        ],
        out_shape=[
            jax.ShapeDtypeStruct((batch, out_dim), jnp.float32),
            jax.ShapeDtypeStruct((batch, 16), jnp.float32),
        ],
        compiler_params=pltpu.CompilerParams(
            dimension_semantics=("parallel",)
        ),
    )(x, wt, b2)


def _sc_mask(kc, t, batch, out_dim):
    rows_per_w = batch // _NW
    chunks = rows_per_w // _CH
    mesh = plsc.VectorSubcoreMesh(
        core_axis_name="c", subcore_axis_name="s",
        num_cores=_NC, num_subcores=_NS,
    )

    @functools.partial(
        pl.kernel,
        out_type=jax.ShapeDtypeStruct((batch, out_dim), jnp.float32),
        mesh=mesh,
        scratch_types=(
            [pltpu.VMEM((_CH, out_dim), jnp.float32)] * _NBUF
            + [pltpu.VMEM((rows_per_w, 16), jnp.float32)]
            + [pltpu.SemaphoreType.DMA] * (2 * _NBUF)
        ),
    )
    def k(kc_hbm, t_hbm, out_hbm, *refs):
        bufs = refs[:_NBUF]
        t_v = refs[_NBUF]
        sin = refs[_NBUF + 1:2 * _NBUF + 1]
        sout = refs[2 * _NBUF + 1:]
        wid = lax.axis_index("s") * _NC + lax.axis_index("c")
        row0 = wid * rows_per_w
        pltpu.sync_copy(t_hbm.at[pl.ds(row0, rows_per_w)], t_v)

        def src(c):
            return kc_hbm.at[pl.ds(row0 + c * _CH, _CH)]

        def dst(c):
            return out_hbm.at[pl.ds(row0 + c * _CH, _CH)]

        for p in range(_NBUF - 1):
            pltpu.async_copy(src(p), bufs[p], sin[p])

        def group(g, _):
            for p in range(_NBUF):
                c = g * _NBUF + p
                buf, s_i, s_o = bufs[p], sin[p], sout[p]
                pltpu.make_async_copy(src(c), buf, s_i).wait()
                for r in range(_CH):
                    tvec = t_v[c * _CH + r, :]

                    def slice_body(i, _, r=r, tvec=tvec, buf=buf):
                        for u in range(8):
                            off = i * 128 + u * 16
                            v = buf[r, pl.ds(off, 16)]
                            buf[r, pl.ds(off, 16)] = jnp.where(
                                v >= tvec, v, jnp.zeros((16,), jnp.float32))
                        return 0

                    lax.fori_loop(0, out_dim // 128, slice_body, 0)
                pltpu.async_copy(buf, dst(c), s_o)
                q = (p + _NBUF - 1) % _NBUF
                nc = c + _NBUF - 1

                @pl.when(jnp.logical_and(nc < chunks, nc >= _NBUF))
                def _(q=q, c=c):
                    pltpu.make_async_copy(bufs[q], dst(c - 1), sout[q]).wait()

                @pl.when(nc < chunks)
                def _(q=q, nc=nc):
                    pltpu.async_copy(src(nc), bufs[q], sin[q])
            return 0

        lax.fori_loop(0, chunks // _NBUF, group, 0)
        for p in range(_NBUF):
            pltpu.make_async_copy(bufs[p], dst(p), sout[p]).wait()

    return k(kc, t)


@jax.jit
def kernel(x, W, b):
    batch, in_dim = x.shape
    out_dim = W.shape[0]
    wt = W.T
    b2 = b.reshape(1, out_dim)
    kc, t = _project_and_threshold(x, wt, b2)
    return _sc_mask(kc, t, batch, out_dim)
